# initial kernel scaffold (unmeasured)
import jax
import jax.numpy as jnp
from jax import lax
from jax.experimental import pallas as pl
from jax.experimental.pallas import tpu as pltpu


def kernel(
    x,
):
    def body(*refs):
        pass

    out_shape = jax.ShapeDtypeStruct(..., jnp.float32)
    return pl.pallas_call(body, out_shape=out_shape)(...)


# baseline (device time: 157603 ns/iter reference)
import jax
import jax.numpy as jnp
from jax import lax
from jax.experimental import pallas as pl
from jax.experimental.pallas import tpu as pltpu

N_DEV = 4


def kernel(x):
    m, n = x.shape
    c = m // N_DEV

    def body(x_ref, out_ref, rs_send, rs_recv, send_sems, recv_sems):
        my = lax.axis_index("i")
        left = (my - 1) % N_DEV
        right = (my + 1) % N_DEV

        barrier_sem = pltpu.get_barrier_semaphore()
        for nbr in (left, right):
            pl.semaphore_signal(
                barrier_sem, inc=1,
                device_id=(nbr,), device_id_type=pl.DeviceIdType.MESH,
            )
        pl.semaphore_wait(barrier_sem, 2)

        rs_send[0, :, :] = x_ref[pl.ds((my % N_DEV) * c, c), :].astype(
            jnp.bfloat16
        )
        for s in range(N_DEV - 1):
            rdma = pltpu.make_async_remote_copy(
                src_ref=rs_send.at[s],
                dst_ref=rs_recv.at[s],
                send_sem=send_sems.at[s],
                recv_sem=recv_sems.at[s],
                device_id=(right,),
                device_id_type=pl.DeviceIdType.MESH,
            )
            rdma.start()
            rdma.wait()
            recv_idx = (my - s - 1) % N_DEV
            acc = rs_recv[s, :, :] + x_ref[pl.ds(recv_idx * c, c), :].astype(
                jnp.bfloat16
            )
            if s < N_DEV - 2:
                rs_send[s + 1, :, :] = acc
            else:
                out_ref[pl.ds(((my + 1) % N_DEV) * c, c), :] = acc

        for s in range(N_DEV - 1):
            idx = (my + 1 - s) % N_DEV
            rdma = pltpu.make_async_remote_copy(
                src_ref=out_ref.at[pl.ds(idx * c, c), :],
                dst_ref=out_ref.at[pl.ds(idx * c, c), :],
                send_sem=send_sems.at[(N_DEV - 1) + s],
                recv_sem=recv_sems.at[(N_DEV - 1) + s],
                device_id=(right,),
                device_id_type=pl.DeviceIdType.MESH,
            )
            rdma.start()
            rdma.wait()

    return pl.pallas_call(
        body,
        out_shape=jax.ShapeDtypeStruct((m, n), jnp.bfloat16),
        in_specs=[pl.BlockSpec(memory_space=pltpu.VMEM)],
        out_specs=pl.BlockSpec(memory_space=pltpu.VMEM),
        scratch_shapes=[
            pltpu.VMEM((N_DEV - 1, c, n), jnp.bfloat16),
            pltpu.VMEM((N_DEV - 1, c, n), jnp.bfloat16),
            pltpu.SemaphoreType.DMA((2 * (N_DEV - 1),)),
            pltpu.SemaphoreType.DMA((2 * (N_DEV - 1),)),
        ],
        compiler_params=pltpu.CompilerParams(collective_id=0),
    )(x)


# device time: 86945 ns/iter; 1.8127x vs baseline; 1.8127x over previous
import jax
import jax.numpy as jnp
from jax import lax
from jax.experimental import pallas as pl
from jax.experimental.pallas import tpu as pltpu

N_DEV = 4
BF16 = jnp.bfloat16


def kernel(x):
    m, n = x.shape
    h = m // 2
    q = m // 4
    e = m // 8

    def body(x_ref, out_ref, s1t, s1b, s2t, s2b, r1t, r1b, r2t, r2b,
             send_sems, recv_sems):
        my = lax.axis_index("i")
        xb = my // 2
        yb = (my % 2) ^ xb
        p1 = my ^ 1
        p2 = 3 - my

        barrier_sem = pltpu.get_barrier_semaphore()
        for nbr in (p1, p2):
            pl.semaphore_signal(
                barrier_sem, inc=1,
                device_id=(nbr,), device_id_type=pl.DeviceIdType.MESH,
            )
        pl.semaphore_wait(barrier_sem, 2)

        def xchg(slot, src, dst, peer):
            rdma = pltpu.make_async_remote_copy(
                src_ref=src, dst_ref=dst,
                send_sem=send_sems.at[slot], recv_sem=recv_sems.at[slot],
                device_id=(peer,), device_id_type=pl.DeviceIdType.MESH,
            )
            rdma.start()
            return rdma

        s1t[...] = x_ref[pl.ds((1 - yb) * q, q), :].astype(BF16)
        s1b[...] = x_ref[pl.ds(h + (1 - xb) * q, q), :].astype(BF16)
        c1t = xchg(0, s1t, r1t, p1)
        c1b = xchg(1, s1b, r1b, p2)
        c1t.wait()
        c1b.wait()

        s2t[...] = r1t[pl.ds((1 - xb) * e, e), :] + x_ref[
            pl.ds(yb * q + (1 - xb) * e, e), :
        ].astype(BF16)
        s2b[...] = r1b[pl.ds((1 - yb) * e, e), :] + x_ref[
            pl.ds(h + xb * q + (1 - yb) * e, e), :
        ].astype(BF16)
        c2t = xchg(2, s2t, r2t, p2)
        c2b = xchg(3, s2b, r2b, p1)
        c2t.wait()
        c2b.wait()
        g_t = yb * q + xb * e
        g_b = h + xb * q + yb * e
        out_ref[pl.ds(g_t, e), :] = (
            r2t[...]
            + r1t[pl.ds(xb * e, e), :]
            + x_ref[pl.ds(g_t, e), :].astype(BF16)
        )
        out_ref[pl.ds(g_b, e), :] = (
            r2b[...]
            + r1b[pl.ds(yb * e, e), :]
            + x_ref[pl.ds(g_b, e), :].astype(BF16)
        )

        c3t = xchg(4, out_ref.at[pl.ds(g_t, e), :],
                   out_ref.at[pl.ds(g_t, e), :], p2)
        c3b = xchg(5, out_ref.at[pl.ds(g_b, e), :],
                    out_ref.at[pl.ds(g_b, e), :], p1)
        c3t.wait()
        c3b.wait()

        c4t = xchg(6, out_ref.at[pl.ds(yb * q, q), :],
                   out_ref.at[pl.ds(yb * q, q), :], p1)
        c4b = xchg(7, out_ref.at[pl.ds(h + xb * q, q), :],
                    out_ref.at[pl.ds(h + xb * q, q), :], p2)
        c4t.wait()
        c4b.wait()

    return pl.pallas_call(
        body,
        out_shape=jax.ShapeDtypeStruct((m, n), BF16),
        in_specs=[pl.BlockSpec(memory_space=pltpu.VMEM)],
        out_specs=pl.BlockSpec(memory_space=pltpu.VMEM),
        scratch_shapes=[
            pltpu.VMEM((q, n), BF16),
            pltpu.VMEM((q, n), BF16),
            pltpu.VMEM((e, n), BF16),
            pltpu.VMEM((e, n), BF16),
            pltpu.VMEM((q, n), BF16),
            pltpu.VMEM((q, n), BF16),
            pltpu.VMEM((e, n), BF16),
            pltpu.VMEM((e, n), BF16),
            pltpu.SemaphoreType.DMA((8,)),
            pltpu.SemaphoreType.DMA((8,)),
        ],
        compiler_params=pltpu.CompilerParams(collective_id=0),
    )(x)


# device time: 85348 ns/iter; 1.8466x vs baseline; 1.0187x over previous
import jax
import jax.numpy as jnp
from jax import lax
from jax.experimental import pallas as pl
from jax.experimental.pallas import tpu as pltpu

N_DEV = 4
BF16 = jnp.bfloat16


def kernel(x):
    m, n = x.shape
    h = m // 2
    q = m // 4
    e = m // 8

    def body(x_ref, out_ref, s1t, s1b, s2t, s2b, r1t, r1b, r2t, r2b,
             send_sems, recv_sems):
        my = lax.axis_index("i")
        xb = my // 2
        yb = (my % 2) ^ xb
        p1 = my ^ 1
        p2 = 3 - my

        a_t = (1 - xb) * e
        b_t = xb * e
        a_b = (1 - yb) * e
        b_b = yb * e
        g_t = yb * q + b_t
        g_b = h + xb * q + b_b
        o_t = yb * q + a_t
        o_b = h + xb * q + a_b

        barrier_sem = pltpu.get_barrier_semaphore()
        for nbr in (p1, p2):
            pl.semaphore_signal(
                barrier_sem, inc=1,
                device_id=(nbr,), device_id_type=pl.DeviceIdType.MESH,
            )
        pl.semaphore_wait(barrier_sem, 2)

        def xchg(slot, src, dst, peer):
            rdma = pltpu.make_async_remote_copy(
                src_ref=src, dst_ref=dst,
                send_sem=send_sems.at[slot], recv_sem=recv_sems.at[slot],
                device_id=(peer,), device_id_type=pl.DeviceIdType.MESH,
            )
            rdma.start()
            return rdma

        s1t[...] = x_ref[pl.ds((1 - yb) * q, q), :].astype(BF16)
        s1b[...] = x_ref[pl.ds(h + (1 - xb) * q, q), :].astype(BF16)
        c0 = xchg(0, s1t.at[pl.ds(a_t, e), :], r1t.at[pl.ds(a_t, e), :], p1)
        c1 = xchg(1, s1b.at[pl.ds(a_b, e), :], r1b.at[pl.ds(a_b, e), :], p2)
        c2 = xchg(2, s1t.at[pl.ds(b_t, e), :], r1t.at[pl.ds(b_t, e), :], p1)
        c3 = xchg(3, s1b.at[pl.ds(b_b, e), :], r1b.at[pl.ds(b_b, e), :], p2)

        c0.wait_recv()
        s2t[...] = r1t[pl.ds(a_t, e), :] + x_ref[
            pl.ds(yb * q + a_t, e), :
        ].astype(BF16)
        c4 = xchg(4, s2t, r2t, p2)
        c1.wait_recv()
        s2b[...] = r1b[pl.ds(a_b, e), :] + x_ref[
            pl.ds(h + xb * q + a_b, e), :
        ].astype(BF16)
        c5 = xchg(5, s2b, r2b, p1)

        c2.wait_recv()
        c4.wait_recv()
        out_ref[pl.ds(g_t, e), :] = (
            r2t[...] + r1t[pl.ds(b_t, e), :]
            + x_ref[pl.ds(g_t, e), :].astype(BF16)
        )
        c6 = xchg(6, out_ref.at[pl.ds(g_t, e), :],
                  out_ref.at[pl.ds(g_t, e), :], p2)
        c8 = xchg(8, out_ref.at[pl.ds(g_t, e), :],
                  out_ref.at[pl.ds(g_t, e), :], p1)
        c3.wait_recv()
        c5.wait_recv()
        out_ref[pl.ds(g_b, e), :] = (
            r2b[...] + r1b[pl.ds(b_b, e), :]
            + x_ref[pl.ds(g_b, e), :].astype(BF16)
        )
        c7 = xchg(7, out_ref.at[pl.ds(g_b, e), :],
                  out_ref.at[pl.ds(g_b, e), :], p1)
        c9 = xchg(9, out_ref.at[pl.ds(g_b, e), :],
                  out_ref.at[pl.ds(g_b, e), :], p2)

        c6.wait_recv()
        c10 = xchg(10, out_ref.at[pl.ds(o_t, e), :],
                   out_ref.at[pl.ds(o_t, e), :], p1)
        c7.wait_recv()
        c11 = xchg(11, out_ref.at[pl.ds(o_b, e), :],
                   out_ref.at[pl.ds(o_b, e), :], p2)

        c8.wait_recv()
        c9.wait_recv()
        c10.wait_recv()
        c11.wait_recv()
        for c in (c0, c1, c2, c3, c4, c5, c6, c7, c8, c9, c10, c11):
            c.wait_send()

    return pl.pallas_call(
        body,
        out_shape=jax.ShapeDtypeStruct((m, n), BF16),
        in_specs=[pl.BlockSpec(memory_space=pltpu.VMEM)],
        out_specs=pl.BlockSpec(memory_space=pltpu.VMEM),
        scratch_shapes=[
            pltpu.VMEM((q, n), BF16),
            pltpu.VMEM((q, n), BF16),
            pltpu.VMEM((e, n), BF16),
            pltpu.VMEM((e, n), BF16),
            pltpu.VMEM((q, n), BF16),
            pltpu.VMEM((q, n), BF16),
            pltpu.VMEM((e, n), BF16),
            pltpu.VMEM((e, n), BF16),
            pltpu.SemaphoreType.DMA((12,)),
            pltpu.SemaphoreType.DMA((12,)),
        ],
        compiler_params=pltpu.CompilerParams(collective_id=0),
    )(x)


# device time: 82036 ns/iter; 1.9211x vs baseline; 1.0404x over previous
import jax
import jax.numpy as jnp
from jax import lax
from jax.experimental import pallas as pl
from jax.experimental.pallas import tpu as pltpu

N_DEV = 4
BF16 = jnp.bfloat16
C = 256
NSLOT = 24


def kernel(x):
    m, n = x.shape
    h = m // 2
    q = m // 4
    e = m // 8

    def body(x_ref, out_ref, s1t, s1b, s2t, s2b, r1t, r1b, r2t, r2b,
             send_sems, recv_sems):
        my = lax.axis_index("i")
        xb = my // 2
        yb = (my % 2) ^ xb
        p1 = my ^ 1
        p2 = 3 - my

        qt_o = (1 - yb) * q
        qt = yb * q
        a_t = (1 - xb) * e
        b_t = xb * e
        g_t = qt + b_t
        o_t = qt + a_t

        qb_o = h + (1 - xb) * q
        qb = h + xb * q
        a_b = (1 - yb) * e
        b_b = yb * e
        g_b = qb + b_b
        o_b = qb + a_b

        barrier_sem = pltpu.get_barrier_semaphore()
        for nbr in (p1, p2):
            pl.semaphore_signal(
                barrier_sem, inc=1,
                device_id=(nbr,), device_id_type=pl.DeviceIdType.MESH,
            )
        pl.semaphore_wait(barrier_sem, 2)

        cs = [None] * NSLOT

        def start(slot, src, dst, peer):
            rdma = pltpu.make_async_remote_copy(
                src_ref=src, dst_ref=dst,
                send_sem=send_sems.at[slot], recv_sem=recv_sems.at[slot],
                device_id=(peer,), device_id_type=pl.DeviceIdType.MESH,
            )
            rdma.start()
            cs[slot] = rdma

        def ck(ref, off, j):
            return ref.at[pl.ds(off + j * C, C), :]

        s1t[...] = x_ref[pl.ds(qt_o, q), :].astype(BF16)
        s1b[...] = x_ref[pl.ds(qb_o, q), :].astype(BF16)

        start(0, ck(s1t, a_t, 0), ck(r1t, a_t, 0), p1)
        start(1, ck(s1b, a_b, 0), ck(r1b, a_b, 0), p2)
        start(2, ck(s1t, a_t, 1), ck(r1t, a_t, 1), p1)
        start(3, ck(s1b, a_b, 1), ck(r1b, a_b, 1), p2)

        cs[0].wait_recv()
        s2t[pl.ds(0, C), :] = r1t[pl.ds(a_t, C), :] + x_ref[
            pl.ds(qt + a_t, C), :
        ].astype(BF16)
        start(4, ck(s2t, 0, 0), ck(r2t, 0, 0), p2)
        cs[1].wait_recv()
        s2b[pl.ds(0, C), :] = r1b[pl.ds(a_b, C), :] + x_ref[
            pl.ds(qb + a_b, C), :
        ].astype(BF16)
        start(5, ck(s2b, 0, 0), ck(r2b, 0, 0), p1)
        cs[2].wait_recv()
        s2t[pl.ds(C, C), :] = r1t[pl.ds(a_t + C, C), :] + x_ref[
            pl.ds(qt + a_t + C, C), :
        ].astype(BF16)
        start(6, ck(s2t, 0, 1), ck(r2t, 0, 1), p2)
        cs[3].wait_recv()
        s2b[pl.ds(C, C), :] = r1b[pl.ds(a_b + C, C), :] + x_ref[
            pl.ds(qb + a_b + C, C), :
        ].astype(BF16)
        start(7, ck(s2b, 0, 1), ck(r2b, 0, 1), p1)

        start(8, ck(s1t, b_t, 0), ck(r1t, b_t, 0), p1)
        start(9, ck(s1b, b_b, 0), ck(r1b, b_b, 0), p2)
        start(10, ck(s1t, b_t, 1), ck(r1t, b_t, 1), p1)
        start(11, ck(s1b, b_b, 1), ck(r1b, b_b, 1), p2)

        cs[4].wait_recv()
        cs[8].wait_recv()
        out_ref[pl.ds(g_t, C), :] = (
            r2t[pl.ds(0, C), :] + r1t[pl.ds(b_t, C), :]
            + x_ref[pl.ds(g_t, C), :].astype(BF16)
        )
        start(12, ck(out_ref, g_t, 0), ck(out_ref, g_t, 0), p2)
        cs[5].wait_recv()
        cs[9].wait_recv()
        out_ref[pl.ds(g_b, C), :] = (
            r2b[pl.ds(0, C), :] + r1b[pl.ds(b_b, C), :]
            + x_ref[pl.ds(g_b, C), :].astype(BF16)
        )
        start(13, ck(out_ref, g_b, 0), ck(out_ref, g_b, 0), p1)
        start(14, ck(out_ref, g_t, 0), ck(out_ref, g_t, 0), p1)
        start(15, ck(out_ref, g_b, 0), ck(out_ref, g_b, 0), p2)
        cs[6].wait_recv()
        cs[10].wait_recv()
        out_ref[pl.ds(g_t + C, C), :] = (
            r2t[pl.ds(C, C), :] + r1t[pl.ds(b_t + C, C), :]
            + x_ref[pl.ds(g_t + C, C), :].astype(BF16)
        )
        start(16, ck(out_ref, g_t, 1), ck(out_ref, g_t, 1), p2)
        cs[7].wait_recv()
        cs[11].wait_recv()
        out_ref[pl.ds(g_b + C, C), :] = (
            r2b[pl.ds(C, C), :] + r1b[pl.ds(b_b + C, C), :]
            + x_ref[pl.ds(g_b + C, C), :].astype(BF16)
        )
        start(17, ck(out_ref, g_b, 1), ck(out_ref, g_b, 1), p1)
        start(18, ck(out_ref, g_t, 1), ck(out_ref, g_t, 1), p1)
        start(19, ck(out_ref, g_b, 1), ck(out_ref, g_b, 1), p2)

        cs[12].wait_recv()
        start(20, ck(out_ref, o_t, 0), ck(out_ref, o_t, 0), p1)
        cs[13].wait_recv()
        start(21, ck(out_ref, o_b, 0), ck(out_ref, o_b, 0), p2)
        cs[16].wait_recv()
        start(22, ck(out_ref, o_t, 1), ck(out_ref, o_t, 1), p1)
        cs[17].wait_recv()
        start(23, ck(out_ref, o_b, 1), ck(out_ref, o_b, 1), p2)

        for slot in (14, 15, 18, 19, 20, 21, 22, 23):
            cs[slot].wait_recv()
        for slot in range(NSLOT):
            cs[slot].wait_send()

    return pl.pallas_call(
        body,
        out_shape=jax.ShapeDtypeStruct((m, n), BF16),
        in_specs=[pl.BlockSpec(memory_space=pltpu.VMEM)],
        out_specs=pl.BlockSpec(memory_space=pltpu.VMEM),
        scratch_shapes=[
            pltpu.VMEM((q, n), BF16),
            pltpu.VMEM((q, n), BF16),
            pltpu.VMEM((e, n), BF16),
            pltpu.VMEM((e, n), BF16),
            pltpu.VMEM((q, n), BF16),
            pltpu.VMEM((q, n), BF16),
            pltpu.VMEM((e, n), BF16),
            pltpu.VMEM((e, n), BF16),
            pltpu.SemaphoreType.DMA((NSLOT,)),
            pltpu.SemaphoreType.DMA((NSLOT,)),
        ],
        compiler_params=pltpu.CompilerParams(collective_id=0),
    )(x)
